# baseline (speedup 1.0000x reference)
import jax
import jax.numpy as jnp
from jax.experimental import pallas as pl
from jax.experimental.pallas import tpu as pltpu
import jax.experimental.pallas.tpu_sc as plsc

L = 160
DIN = L * 8
BB = 1024
GW = 256
TABW = 484 * 8


def _sc_gather_feat(comb, fused_flat):
    M = comb.shape[0]
    mesh = plsc.VectorSubcoreMesh(core_axis_name="core",
                                  subcore_axis_name="subcore")

    @pl.kernel(out_type=jax.ShapeDtypeStruct((8, M), jnp.float32), mesh=mesh,
               compiler_params=pltpu.CompilerParams(needs_layout_passes=False))
    def k(tab_hbm, i_hbm, o_hbm):
        def body(tab_v, i_vmem, o_vmem):
            @pl.loop(0, GW, step=16)
            def _(kk):
                base = i_vmem[pl.ds(kk, 16)] * 8
                for j in range(8):
                    o_vmem[j, pl.ds(kk, 16)] = plsc.load_gather(
                        tab_v, [base + j])

        pltpu.emit_pipeline(
            body,
            grid=(M // GW,),
            in_specs=[pl.BlockSpec((TABW,), index_map=lambda i: (0,)),
                      pl.BlockSpec((GW,), index_map=lambda i: (i,))],
            out_specs=[pl.BlockSpec((8, GW), index_map=lambda i: (0, i))],
            core_axis_name=("core", "subcore"),
            dimension_semantics=(pltpu.PARALLEL,),
        )(tab_hbm, i_hbm, o_hbm)

    return k(fused_flat, comb)


def _mlp_kernel(feat8_ref, w1_ref, b1_ref, w2_ref, b2_ref, out_ref, feat_ref):
    for j in range(8):
        feat_ref[:, j * L:(j + 1) * L] = feat8_ref[j].astype(jnp.bfloat16)
    feat = feat_ref[...]
    h = jnp.dot(feat, w1_ref[...], preferred_element_type=jnp.float32)
    h = jnp.maximum(h + b1_ref[...], 0.0).astype(jnp.bfloat16)
    out = jnp.dot(h, w2_ref[...], preferred_element_type=jnp.float32)
    out_ref[...] = out + b2_ref[...]


@jax.jit
def kernel(frac_app_idx, all_true_idx, all_false_idx, frac_tf_idx,
           frac_app_tab, true_tab, false_tab, frac_tf_tab,
           W1, b1, W2, b2):
    B = frac_app_idx.shape[0]
    H2 = W1.shape[1]
    H = W2.shape[1]
    bb = min(BB, B)

    parts = [
        jnp.broadcast_to(frac_app_tab[:, None, None, None, :], (11, 2, 2, 11, 2)),
        jnp.broadcast_to(true_tab[None, :, None, None, :], (11, 2, 2, 11, 2)),
        jnp.broadcast_to(false_tab[None, None, :, None, :], (11, 2, 2, 11, 2)),
        jnp.broadcast_to(frac_tf_tab[None, None, None, :, :], (11, 2, 2, 11, 2)),
    ]
    fused_flat = jnp.concatenate(parts, axis=-1).reshape(TABW)

    comb = (((frac_app_idx * 2 + all_true_idx) * 2 + all_false_idx) * 11
            + frac_tf_idx).reshape(B * L).astype(jnp.int32)

    feat8 = _sc_gather_feat(comb, fused_flat).reshape(8, B, L)

    W1p = (W1.reshape(4, L, 2, H2).transpose(0, 2, 1, 3)
           .reshape(DIN, H2).astype(jnp.bfloat16))
    W2b = W2.astype(jnp.bfloat16)

    out = pl.pallas_call(
        _mlp_kernel,
        grid=(B // bb,),
        in_specs=[
            pl.BlockSpec((8, bb, L), lambda i: (0, i, 0)),
            pl.BlockSpec((DIN, H2), lambda i: (0, 0)),
            pl.BlockSpec((1, H2), lambda i: (0, 0)),
            pl.BlockSpec((H2, H), lambda i: (0, 0)),
            pl.BlockSpec((1, H), lambda i: (0, 0)),
        ],
        out_specs=pl.BlockSpec((bb, H), lambda i: (i, 0)),
        out_shape=jax.ShapeDtypeStruct((B, H), jnp.float32),
        scratch_shapes=[pltpu.VMEM((bb, DIN), jnp.bfloat16)],
    )(feat8, W1p, b1.reshape(1, H2), W2b, b2.reshape(1, H))
    return out
